# baseline (speedup 1.0000x reference)
import functools

import jax
import jax.numpy as jnp
import numpy as np
from jax import lax
from jax.experimental import pallas as pl
from jax.experimental.pallas import tpu as pltpu
from jax.experimental.pallas import tpu_sc as plsc

_N = 10000
_E = 320000
_C = 128
_K = 16
_B = 8
_CPG = 10
_NW = 32
_EPW = _E // _NW
_SUPC = 10
_NSUP = 125

_NP = 10240
_STRIPE = _NP // 16
_SRCW = 128


def _basis_body(a_ref, o_ref):
    a = a_ref[...]
    f = [a[:, d : d + 1] for d in range(4)]
    g = [1.0 - a[:, d : d + 1] for d in range(4)]
    t01 = [g[0] * g[1], f[0] * g[1], g[0] * f[1], f[0] * f[1]]
    t23 = [g[2] * g[3], f[2] * g[3], g[2] * f[3], f[2] * f[3]]
    blk = a.shape[0]
    o_ref[...] = jnp.concatenate(
        [jnp.broadcast_to(t01[k & 3] * t23[k >> 2], (blk, 16))
         for k in range(16)], axis=1
    )


def _basis_tc(edge_attr):
    blk = 3200
    return pl.pallas_call(
        _basis_body,
        grid=(_E // blk,),
        in_specs=[pl.BlockSpec((blk, 4), lambda i: (i, 0))],
        out_specs=pl.BlockSpec((blk, 256), lambda i: (i, 0)),
        out_shape=jax.ShapeDtypeStruct((_E, 256), jnp.float32),
    )(edge_attr)


def _mm_body(x_ref, w1_ref, w2_ref, b_ref, y_ref, r_ref):
    x = x_ref[...]
    y_ref[...] = jnp.dot(
        x, w1_ref[...], preferred_element_type=jnp.float32
    ).astype(jnp.bfloat16)
    r_ref[...] = (
        jnp.dot(x, w2_ref[...], preferred_element_type=jnp.float32) + b_ref[...]
    )


def _mm_tc(x, wflat, root, b):
    blk = 1000
    return pl.pallas_call(
        _mm_body,
        grid=(_N // blk,),
        in_specs=[
            pl.BlockSpec((blk, _C), lambda i: (i, 0)),
            pl.BlockSpec((_C, _K * _C), lambda i: (0, 0)),
            pl.BlockSpec((_C, _C), lambda i: (0, 0)),
            pl.BlockSpec((1, _C), lambda i: (0, 0)),
        ],
        out_specs=[
            pl.BlockSpec((blk, _K * _C), lambda i: (i, 0)),
            pl.BlockSpec((blk, _C), lambda i: (i, 0)),
        ],
        out_shape=[
            jax.ShapeDtypeStruct((_N, _K * _C), jnp.bfloat16),
            jax.ShapeDtypeStruct((_N, _C), jnp.float32),
        ],
    )(x, wflat, root, b.reshape(1, _C))


def _agg_body(y_hbm, srcw_hbm, dstp_hbm, bas_hbm, z_hbm, p_hbm,
              acc_sh, srcbig, rows0, rows1, bsp0, bsp1, ybuf, dstsup, dstbuf,
              sem0, sem1, sem2, sem3):
    c = lax.axis_index("c")
    s = lax.axis_index("s")
    w = c * 16 + s
    off = pl.multiple_of(s * _STRIPE, _STRIPE)

    pltpu.sync_copy(z_hbm.at[s], acc_sh.at[pl.ds(off, _STRIPE)])
    plsc.subcore_barrier()

    rows = (rows0, rows1)
    sems = (sem0, sem1)
    bsps = (bsp0, bsp1)
    bsems = (sem2, sem3)
    c16 = jnp.full((16,), 16, jnp.int32)
    nchk = _NSUP * _SUPC

    pltpu.sync_copy(srcw_hbm.at[w, 0], srcbig.at[pl.ds(0, _SRCW)])
    pltpu.make_async_copy(
        y_hbm.at[srcbig.at[pl.ds(0, _B)]], rows0, sem0).start()
    pltpu.make_async_copy(bas_hbm.at[w, 0], bsp0, sem2).start()

    def sup_body(sup, _):
        par = sup % 2
        base = pl.multiple_of(par * _SRCW, _SRCW)
        nbase = pl.multiple_of((1 - par) * _SRCW, _SRCW)
        pltpu.sync_copy(
            srcw_hbm.at[w, jnp.minimum(sup + 1, _NSUP - 1)],
            srcbig.at[pl.ds(nbase, _SRCW)])
        pltpu.sync_copy(dstp_hbm.at[w, sup], dstsup)

        for ci in range(_SUPC):
            rbuf, sem = rows[ci % 2], sems[ci % 2]
            bspbuf, bsem = bsps[ci % 2], bsems[ci % 2]
            pltpu.make_async_copy(
                y_hbm.at[srcbig.at[pl.ds(base + ci * _B, _B)]],
                rbuf, sem).wait()
            pltpu.make_async_copy(bas_hbm.at[w, 0], bspbuf, bsem).wait()
            pltpu.make_async_copy(
                y_hbm.at[srcbig.at[pl.ds(base + (ci + 1) * _B, _B)]],
                rows[(ci + 1) % 2], sems[(ci + 1) % 2]).start()
            nci = jnp.minimum(sup * _SUPC + ci + 1, nchk - 1)
            pltpu.make_async_copy(
                bas_hbm.at[w, nci], bsps[(ci + 1) % 2],
                bsems[(ci + 1) % 2]).start()

            def edge_body(bb, _, _ci=ci, bspbuf=bspbuf):
                for half in range(2):
                    b = bb * 2 + half
                    sk = [bspbuf[b, pl.ds(k * 16, 16)] for k in range(16)]
                    row = _ci * _B + b
                    for j in range(4):
                        acc_a = None
                        for k in range(16):
                            wrd = rbuf[b, pl.ds(k * 64 + j * 16, 16)]
                            va = lax.bitcast_convert_type(
                                lax.shift_left(wrd, c16), jnp.float32)
                            vb = lax.bitcast_convert_type(wrd, jnp.float32)
                            if acc_a is None:
                                acc_a = sk[0] * va
                                acc_b = sk[0] * vb
                            else:
                                acc_a = acc_a + sk[k] * va
                                acc_b = acc_b + sk[k] * vb
                        ybuf[row, pl.ds(j * 32, 16)] = acc_a
                        ybuf[row, pl.ds(j * 32 + 16, 16)] = acc_b
                return 0

            lax.fori_loop(0, _B // 2, edge_body, 0)

        for tt in range(5):
            dstbuf[0, pl.ds(tt * 16, 16)] = dstsup[0, pl.ds(tt * 16, 16)]
        pltpu.sync_copy(ybuf, acc_sh.at[dstbuf.at[0]], add=True)
        return 0

    lax.fori_loop(0, _NSUP, sup_body, 0)

    pltpu.make_async_copy(
        y_hbm.at[srcbig.at[pl.ds(_SUPC * _B, _B)]], rows0, sem0).wait()
    pltpu.make_async_copy(bas_hbm.at[w, 0], bsp0, sem2).wait()

    plsc.subcore_barrier()
    pltpu.sync_copy(acc_sh.at[pl.ds(off, _STRIPE)], p_hbm.at[c, s])


def _agg_sc(y, srcw, dstp, bas, zer):
    mesh = plsc.VectorSubcoreMesh(core_axis_name="c", subcore_axis_name="s")
    f = pl.kernel(
        _agg_body,
        out_type=jax.ShapeDtypeStruct((2, 16, _STRIPE, _C), jnp.float32),
        mesh=mesh,
        scratch_types=[
            pltpu.VMEM_SHARED((_NP, _C), jnp.float32),
            pltpu.VMEM((2 * _SRCW,), jnp.int32),
            pltpu.VMEM((_B, _K * _C // 2), jnp.int32),
            pltpu.VMEM((_B, _K * _C // 2), jnp.int32),
            pltpu.VMEM((_B, 256), jnp.float32),
            pltpu.VMEM((_B, 256), jnp.float32),
            pltpu.VMEM((_CPG * _B, _C), jnp.float32),
            pltpu.VMEM((1, _C), jnp.int32),
            pltpu.VMEM((1, _CPG * _B), jnp.int32),
            pltpu.SemaphoreType.DMA,
            pltpu.SemaphoreType.DMA,
            pltpu.SemaphoreType.DMA,
            pltpu.SemaphoreType.DMA,
        ],
    )
    return f(y, srcw, dstp, bas, zer).reshape(2, _NP, _C)


def _deg_body(dstp_hbm, z_hbm, d_hbm, acc_sh, dstsup, dstbuf, onebuf):
    c = lax.axis_index("c")
    s = lax.axis_index("s")
    w = c * 16 + s
    off = pl.multiple_of(s * _STRIPE, _STRIPE)

    pltpu.sync_copy(z_hbm.at[s], acc_sh.at[pl.ds(off, _STRIPE)])
    one16 = jnp.full((16,), 1.0, jnp.float32)

    def ones_body(r, _):
        for j in range(8):
            onebuf[r, pl.ds(j * 16, 16)] = one16
        return 0

    lax.fori_loop(0, _CPG * _B, ones_body, 0)
    plsc.subcore_barrier()

    def sup_body(sup, _):
        pltpu.sync_copy(dstp_hbm.at[w, sup], dstsup)
        for tt in range(5):
            dstbuf[0, pl.ds(tt * 16, 16)] = dstsup[0, pl.ds(tt * 16, 16)]
        pltpu.sync_copy(onebuf, acc_sh.at[dstbuf.at[0]], add=True)
        return 0

    lax.fori_loop(0, _NSUP, sup_body, 0)

    plsc.subcore_barrier()
    pltpu.sync_copy(acc_sh.at[pl.ds(off, _STRIPE)], d_hbm.at[c, s])


def _deg_sc(dstp, zer):
    mesh = plsc.VectorSubcoreMesh(core_axis_name="c", subcore_axis_name="s")
    f = pl.kernel(
        _deg_body,
        out_type=jax.ShapeDtypeStruct((2, 16, _STRIPE, _C), jnp.float32),
        mesh=mesh,
        scratch_types=[
            pltpu.VMEM_SHARED((_NP, _C), jnp.float32),
            pltpu.VMEM((1, _C), jnp.int32),
            pltpu.VMEM((1, _CPG * _B), jnp.int32),
            pltpu.VMEM((_CPG * _B, _C), jnp.float32),
        ],
    )
    return f(dstp, zer).reshape(2, _NP, _C)


def _epi_body(p_ref, d_ref, r_ref, o_ref, *, act):
    p = p_ref[...]
    d = d_ref[...]
    acc = p[0] + p[1]
    deg = (d[0] + d[1])[:, 0:1]
    v = acc / jnp.maximum(deg, 1.0) + r_ref[...]
    if act == "elu":
        o_ref[...] = jnp.where(v > 0, v, jnp.exp(v) - 1.0)
    else:
        o_ref[...] = jnp.tanh(v)


def _epi_tc(p, d, r, act):
    blk = 1000
    return pl.pallas_call(
        functools.partial(_epi_body, act=act),
        grid=(_N // blk,),
        in_specs=[
            pl.BlockSpec((2, blk, _C), lambda i: (0, i, 0)),
            pl.BlockSpec((2, blk, _C), lambda i: (0, i, 0)),
            pl.BlockSpec((blk, _C), lambda i: (i, 0)),
        ],
        out_specs=pl.BlockSpec((blk, _C), lambda i: (i, 0)),
        out_shape=jax.ShapeDtypeStruct((_N, _C), jnp.float32),
    )(p, d, r)


def kernel(t, x, edge_index, edge_attr, W_in, root_in, b_in, W_out, root_out, b_out):
    Nb, V, C = x.shape
    xf = x.reshape(Nb * V, C)
    src = edge_index[0].astype(jnp.int32)
    dst = edge_index[1].astype(jnp.int32)

    srcpad = jnp.concatenate(
        [src.reshape(_NW, _EPW), jnp.zeros((_NW, 80), jnp.int32)], axis=1)
    main = srcpad[:, :_EPW].reshape(_NW, _NSUP, _SUPC * _B)
    phant = srcpad[:, _SUPC * _B : _EPW + _SUPC * _B]
    phant = phant.reshape(_NW, _NSUP, _SUPC * _B)[:, :, :_B]
    srcw = jnp.concatenate(
        [main, phant,
         jnp.zeros((_NW, _NSUP, _SRCW - _SUPC * _B - _B), jnp.int32)], axis=2)

    dstp = jnp.concatenate(
        [dst.reshape(_NW, _NSUP, _SUPC * _B),
         jnp.zeros((_NW, _NSUP, _C - _SUPC * _B), jnp.int32)], axis=2)
    dstp = dstp.reshape(_NW, _NSUP, 1, _C)

    basis = _basis_tc(edge_attr).reshape(_NW, _NSUP * _SUPC, _B, 256)
    zer = jnp.zeros((16, _STRIPE, _C), jnp.float32)

    d = _deg_sc(dstp, zer)

    pos = np.arange(_K * _C)
    perm = (pos // 32) * 32 + 16 * (pos % 2) + (pos % 32) // 2
    perm = jnp.asarray(perm)

    w1 = W_in.transpose(1, 0, 2).reshape(_C, _K * _C)[:, perm]
    y1, r1 = _mm_tc(xf, w1, root_in, b_in)
    y1 = lax.bitcast_convert_type(
        y1.reshape(_N, _K * _C // 2, 2), jnp.int32)
    p1 = _agg_sc(y1, srcw, dstp, basis, zer)
    h = _epi_tc(p1, d, r1, "elu")

    w2 = W_out.transpose(1, 0, 2).reshape(_C, _K * _C)[:, perm]
    y2, r2 = _mm_tc(h, w2, root_out, b_out)
    y2 = lax.bitcast_convert_type(
        y2.reshape(_N, _K * _C // 2, 2), jnp.int32)
    p2 = _agg_sc(y2, srcw, dstp, basis, zer)
    o = _epi_tc(p2, d, r2, "tanh")
    return o.reshape(Nb, V, C)


# speedup vs baseline: 1.2198x; 1.2198x over previous
import functools

import jax
import jax.numpy as jnp
import numpy as np
from jax import lax
from jax.experimental import pallas as pl
from jax.experimental.pallas import tpu as pltpu
from jax.experimental.pallas import tpu_sc as plsc

_N = 10000
_E = 320000
_C = 128
_K = 16
_B = 8
_CPG = 10
_NW = 32
_EPW = _E // _NW
_SUPC = 10
_NSUP = 125

_NP = 10240
_STRIPE = _NP // 16
_SRCW = 128


def _basis_body(a_ref, o_ref):
    a = a_ref[...]
    f = [a[:, d : d + 1] for d in range(4)]
    g = [1.0 - a[:, d : d + 1] for d in range(4)]
    t01 = [g[0] * g[1], f[0] * g[1], g[0] * f[1], f[0] * f[1]]
    t23 = [g[2] * g[3], f[2] * g[3], g[2] * f[3], f[2] * f[3]]
    blk = a.shape[0]
    o_ref[...] = jnp.concatenate(
        [jnp.broadcast_to(t01[k & 3] * t23[k >> 2], (blk, 16))
         for k in range(16)], axis=1
    )


def _basis_tc(edge_attr):
    blk = 3200
    return pl.pallas_call(
        _basis_body,
        grid=(_E // blk,),
        in_specs=[pl.BlockSpec((blk, 4), lambda i: (i, 0))],
        out_specs=pl.BlockSpec((blk, 256), lambda i: (i, 0)),
        out_shape=jax.ShapeDtypeStruct((_E, 256), jnp.float32),
    )(edge_attr)


def _mm_body(x_ref, w1_ref, w2_ref, b_ref, y_ref, r_ref):
    x = x_ref[...]
    y_ref[...] = jnp.dot(
        x, w1_ref[...], preferred_element_type=jnp.float32
    ).astype(jnp.bfloat16)
    r_ref[...] = (
        jnp.dot(x, w2_ref[...], preferred_element_type=jnp.float32) + b_ref[...]
    )


def _mm_tc(x, wflat, root, b):
    blk = 1000
    return pl.pallas_call(
        _mm_body,
        grid=(_N // blk,),
        in_specs=[
            pl.BlockSpec((blk, _C), lambda i: (i, 0)),
            pl.BlockSpec((_C, _K * _C), lambda i: (0, 0)),
            pl.BlockSpec((_C, _C), lambda i: (0, 0)),
            pl.BlockSpec((1, _C), lambda i: (0, 0)),
        ],
        out_specs=[
            pl.BlockSpec((blk, _K * _C), lambda i: (i, 0)),
            pl.BlockSpec((blk, _C), lambda i: (i, 0)),
        ],
        out_shape=[
            jax.ShapeDtypeStruct((_N, _K * _C), jnp.bfloat16),
            jax.ShapeDtypeStruct((_N, _C), jnp.float32),
        ],
    )(x, wflat, root, b.reshape(1, _C))


def _agg_body(y_hbm, srcw_hbm, dstp_hbm, bas_hbm, z_hbm, p_hbm,
              acc_sh, srcbig, rows0, rows1, bsp0, bsp1, ybuf, dstsup, dstbuf,
              sem0, sem1, sem2, sem3):
    c = lax.axis_index("c")
    s = lax.axis_index("s")
    w = c * 16 + s
    off = pl.multiple_of(s * _STRIPE, _STRIPE)

    pltpu.sync_copy(z_hbm.at[s], acc_sh.at[pl.ds(off, _STRIPE)])
    plsc.subcore_barrier()

    rows = (rows0, rows1)
    sems = (sem0, sem1)
    bsps = (bsp0, bsp1)
    bsems = (sem2, sem3)
    c16 = jnp.full((16,), 16, jnp.int32)
    nchk = _NSUP * _SUPC

    pltpu.sync_copy(srcw_hbm.at[w, 0], srcbig.at[pl.ds(0, _SRCW)])
    pltpu.make_async_copy(
        y_hbm.at[srcbig.at[pl.ds(0, _B)]], rows0, sem0).start()
    pltpu.make_async_copy(bas_hbm.at[w, 0], bsp0, sem2).start()

    def sup_body(sup, _):
        par = sup % 2
        base = pl.multiple_of(par * _SRCW, _SRCW)
        nbase = pl.multiple_of((1 - par) * _SRCW, _SRCW)
        pltpu.sync_copy(
            srcw_hbm.at[w, jnp.minimum(sup + 1, _NSUP - 1)],
            srcbig.at[pl.ds(nbase, _SRCW)])
        pltpu.sync_copy(dstp_hbm.at[w, sup], dstsup)

        for ci in range(_SUPC):
            rbuf, sem = rows[ci % 2], sems[ci % 2]
            bspbuf, bsem = bsps[ci % 2], bsems[ci % 2]
            pltpu.make_async_copy(
                y_hbm.at[srcbig.at[pl.ds(base + ci * _B, _B)]],
                rbuf, sem).wait()
            pltpu.make_async_copy(bas_hbm.at[w, 0], bspbuf, bsem).wait()
            pltpu.make_async_copy(
                y_hbm.at[srcbig.at[pl.ds(base + (ci + 1) * _B, _B)]],
                rows[(ci + 1) % 2], sems[(ci + 1) % 2]).start()
            nci = jnp.minimum(sup * _SUPC + ci + 1, nchk - 1)
            pltpu.make_async_copy(
                bas_hbm.at[w, nci], bsps[(ci + 1) % 2],
                bsems[(ci + 1) % 2]).start()

            def edge_body(b, _, _ci=ci, bspbuf=bspbuf):
                sk = [bspbuf[b, pl.ds(k * 16, 16)] for k in range(16)]
                row = _ci * _B + b
                for j in range(4):
                    acc_a = None
                    for k in range(16):
                        wrd = rbuf[b, pl.ds(k * 64 + j * 16, 16)]
                        va = lax.bitcast_convert_type(
                            lax.shift_left(wrd, c16), jnp.float32)
                        vb = lax.bitcast_convert_type(wrd, jnp.float32)
                        if acc_a is None:
                            acc_a = sk[0] * va
                            acc_b = sk[0] * vb
                        else:
                            acc_a = acc_a + sk[k] * va
                            acc_b = acc_b + sk[k] * vb
                    ybuf[row, pl.ds(j * 32, 16)] = acc_a
                    ybuf[row, pl.ds(j * 32 + 16, 16)] = acc_b
                return 0

            lax.fori_loop(0, _B, edge_body, 0)

        for tt in range(5):
            dstbuf[0, pl.ds(tt * 16, 16)] = dstsup[0, pl.ds(tt * 16, 16)]
        pltpu.sync_copy(ybuf, acc_sh.at[dstbuf.at[0]], add=True)
        return 0

    lax.fori_loop(0, _NSUP, sup_body, 0)

    pltpu.make_async_copy(
        y_hbm.at[srcbig.at[pl.ds(_SUPC * _B, _B)]], rows0, sem0).wait()
    pltpu.make_async_copy(bas_hbm.at[w, 0], bsp0, sem2).wait()

    plsc.subcore_barrier()
    pltpu.sync_copy(acc_sh.at[pl.ds(off, _STRIPE)], p_hbm.at[c, s])


def _agg_sc(y, srcw, dstp, bas, zer):
    mesh = plsc.VectorSubcoreMesh(core_axis_name="c", subcore_axis_name="s")
    f = pl.kernel(
        _agg_body,
        out_type=jax.ShapeDtypeStruct((2, 16, _STRIPE, _C), jnp.float32),
        mesh=mesh,
        scratch_types=[
            pltpu.VMEM_SHARED((_NP, _C), jnp.float32),
            pltpu.VMEM((2 * _SRCW,), jnp.int32),
            pltpu.VMEM((_B, _K * _C // 2), jnp.int32),
            pltpu.VMEM((_B, _K * _C // 2), jnp.int32),
            pltpu.VMEM((_B, 256), jnp.float32),
            pltpu.VMEM((_B, 256), jnp.float32),
            pltpu.VMEM((_CPG * _B, _C), jnp.float32),
            pltpu.VMEM((1, _C), jnp.int32),
            pltpu.VMEM((1, _CPG * _B), jnp.int32),
            pltpu.SemaphoreType.DMA,
            pltpu.SemaphoreType.DMA,
            pltpu.SemaphoreType.DMA,
            pltpu.SemaphoreType.DMA,
        ],
    )
    return f(y, srcw, dstp, bas, zer).reshape(2, _NP, _C)


def _deg_body(dstp_hbm, z_hbm, d_hbm, acc_sh, dstsup, dstbuf, onebuf):
    c = lax.axis_index("c")
    s = lax.axis_index("s")
    w = c * 16 + s
    off = pl.multiple_of(s * _STRIPE, _STRIPE)

    pltpu.sync_copy(z_hbm.at[s], acc_sh.at[pl.ds(off, _STRIPE)])
    one16 = jnp.full((16,), 1.0, jnp.float32)

    def ones_body(r, _):
        for j in range(8):
            onebuf[r, pl.ds(j * 16, 16)] = one16
        return 0

    lax.fori_loop(0, _CPG * _B, ones_body, 0)
    plsc.subcore_barrier()

    def sup_body(sup, _):
        pltpu.sync_copy(dstp_hbm.at[w, sup], dstsup)
        for tt in range(5):
            dstbuf[0, pl.ds(tt * 16, 16)] = dstsup[0, pl.ds(tt * 16, 16)]
        pltpu.sync_copy(onebuf, acc_sh.at[dstbuf.at[0]], add=True)
        return 0

    lax.fori_loop(0, _NSUP, sup_body, 0)

    plsc.subcore_barrier()
    pltpu.sync_copy(acc_sh.at[pl.ds(off, _STRIPE)], d_hbm.at[c, s])


def _deg_sc(dstp, zer):
    mesh = plsc.VectorSubcoreMesh(core_axis_name="c", subcore_axis_name="s")
    f = pl.kernel(
        _deg_body,
        out_type=jax.ShapeDtypeStruct((2, 16, _STRIPE, _C), jnp.float32),
        mesh=mesh,
        scratch_types=[
            pltpu.VMEM_SHARED((_NP, _C), jnp.float32),
            pltpu.VMEM((1, _C), jnp.int32),
            pltpu.VMEM((1, _CPG * _B), jnp.int32),
            pltpu.VMEM((_CPG * _B, _C), jnp.float32),
        ],
    )
    return f(dstp, zer).reshape(2, _NP, _C)


def _epi_body(p_ref, d_ref, r_ref, o_ref, *, act):
    p = p_ref[...]
    d = d_ref[...]
    acc = p[0] + p[1]
    deg = (d[0] + d[1])[:, 0:1]
    v = acc / jnp.maximum(deg, 1.0) + r_ref[...]
    if act == "elu":
        o_ref[...] = jnp.where(v > 0, v, jnp.exp(v) - 1.0)
    else:
        o_ref[...] = jnp.tanh(v)


def _epi_tc(p, d, r, act):
    blk = 1000
    return pl.pallas_call(
        functools.partial(_epi_body, act=act),
        grid=(_N // blk,),
        in_specs=[
            pl.BlockSpec((2, blk, _C), lambda i: (0, i, 0)),
            pl.BlockSpec((2, blk, _C), lambda i: (0, i, 0)),
            pl.BlockSpec((blk, _C), lambda i: (i, 0)),
        ],
        out_specs=pl.BlockSpec((blk, _C), lambda i: (i, 0)),
        out_shape=jax.ShapeDtypeStruct((_N, _C), jnp.float32),
    )(p, d, r)


def kernel(t, x, edge_index, edge_attr, W_in, root_in, b_in, W_out, root_out, b_out):
    Nb, V, C = x.shape
    xf = x.reshape(Nb * V, C)
    src = edge_index[0].astype(jnp.int32)
    dst = edge_index[1].astype(jnp.int32)

    srcpad = jnp.concatenate(
        [src.reshape(_NW, _EPW), jnp.zeros((_NW, 80), jnp.int32)], axis=1)
    main = srcpad[:, :_EPW].reshape(_NW, _NSUP, _SUPC * _B)
    phant = srcpad[:, _SUPC * _B : _EPW + _SUPC * _B]
    phant = phant.reshape(_NW, _NSUP, _SUPC * _B)[:, :, :_B]
    srcw = jnp.concatenate(
        [main, phant,
         jnp.zeros((_NW, _NSUP, _SRCW - _SUPC * _B - _B), jnp.int32)], axis=2)

    dstp = jnp.concatenate(
        [dst.reshape(_NW, _NSUP, _SUPC * _B),
         jnp.zeros((_NW, _NSUP, _C - _SUPC * _B), jnp.int32)], axis=2)
    dstp = dstp.reshape(_NW, _NSUP, 1, _C)

    basis = _basis_tc(edge_attr).reshape(_NW, _NSUP * _SUPC, _B, 256)
    zer = jnp.zeros((16, _STRIPE, _C), jnp.float32)

    d = _deg_sc(dstp, zer)

    pos = np.arange(_K * _C)
    perm = (pos // 32) * 32 + 16 * (pos % 2) + (pos % 32) // 2
    perm = jnp.asarray(perm)

    w1 = W_in.transpose(1, 0, 2).reshape(_C, _K * _C)[:, perm]
    y1, r1 = _mm_tc(xf, w1, root_in, b_in)
    y1 = lax.bitcast_convert_type(
        y1.reshape(_N, _K * _C // 2, 2), jnp.int32)
    p1 = _agg_sc(y1, srcw, dstp, basis, zer)
    h = _epi_tc(p1, d, r1, "elu")

    w2 = W_out.transpose(1, 0, 2).reshape(_C, _K * _C)[:, perm]
    y2, r2 = _mm_tc(h, w2, root_out, b_out)
    y2 = lax.bitcast_convert_type(
        y2.reshape(_N, _K * _C // 2, 2), jnp.int32)
    p2 = _agg_sc(y2, srcw, dstp, basis, zer)
    o = _epi_tc(p2, d, r2, "tanh")
    return o.reshape(Nb, V, C)
